# Initial kernel scaffold; baseline (speedup 1.0000x reference)
import jax
import jax.numpy as jnp
from jax.experimental import pallas as pl


def kernel(x, Wb0, bb0, Wb1, bb1, Wb2, bb2, Wcb, bcb, Wch, bch, Wlb, blb, Wlh, blh):
    raise NotImplementedError("write your pallas kernel here")


# speedup vs baseline: 1.6997x; 1.6997x over previous
import jax
import jax.numpy as jnp
import numpy as np
from jax.experimental import pallas as pl
from jax.experimental.pallas import tpu as pltpu

B = 256
MAX_NODES = 1023
N_FEAT = 32
HID = 64
LEAF_IDX = 0
FEAT_IDX = 2046
THR_IDX = 2078
OP_IDX = 2088
EOS_IDX = 4134
P_DIM = 4135

BB = 8


def _tree_norms():
    n = MAX_NODES
    src, dst = [], []
    for i in range(1, n):
        p = (i - 1) // 2
        src.extend([i, p])
        dst.extend([p, i])
    src.extend(range(n))
    dst.extend(range(n))
    src = np.asarray(src, np.int64)
    dst = np.asarray(dst, np.int64)
    deg = np.bincount(dst, minlength=n).astype(np.float64)
    w = 1.0 / np.sqrt(deg[src] * deg[dst])

    ns = np.zeros(n)
    npar = np.zeros(n)
    nl = np.zeros(n)
    nr = np.zeros(n)
    for s, d, ww in zip(src, dst, w):
        if s == d:
            ns[d] += ww
        elif s == 2 * d + 1:
            nl[d] += ww
        elif s == 2 * d + 2:
            nr[d] += ww
        elif d >= 1 and s == (d - 1) // 2:
            npar[d] += ww
        else:
            raise AssertionError("unexpected edge")
    return ns, npar, nl, nr


_NS, _NPAR, _NL, _NR = _tree_norms()

NS_COL = np.zeros((MAX_NODES + 1, 1), np.float32)
NS_COL[:MAX_NODES, 0] = _NS
NL_COL = _NL[:511].astype(np.float32).reshape(511, 1)
NR_COL = _NR[:511].astype(np.float32).reshape(511, 1)
NPAR_COL = _NPAR[1:1023].astype(np.float32).reshape(1022, 1)


def _aggregate(h, norms):
    ns, nl, nr, npp = norms
    f = h.shape[1]
    agg = ns[...] * h
    pairs = h[1:1023].reshape(511, 2, f)
    child = nl[...] * pairs[:, 0, :] + nr[...] * pairs[:, 1, :]
    agg = agg + jnp.concatenate(
        [child, jnp.zeros((513, f), jnp.float32)], axis=0)
    par = jnp.broadcast_to(h[0:511][:, None, :], (511, 2, f)).reshape(1022, f)
    par = npp[...] * par
    agg = agg + jnp.concatenate(
        [jnp.zeros((1, f), jnp.float32), par,
         jnp.zeros((1, f), jnp.float32)], axis=0)
    return agg


def _leaky(x):
    return jnp.where(x >= 0, x, 0.01 * x)


def _gcn(h, w_ref, b_ref, norms):
    hw = jnp.dot(h, w_ref[...], preferred_element_type=jnp.float32)
    return _aggregate(hw, norms) + b_ref[...]


def _head_flat(hb, wrep_ref, brep_ref, norms):
    aggc = _aggregate(hb, norms)
    rep = jnp.broadcast_to(aggc[:, None, :], (1024, 2, HID)).reshape(2048, HID)
    prod = rep[:2046] * wrep_ref[...]
    return jnp.sum(prod, axis=1, keepdims=True) + brep_ref[...]


def _kernel_body(x_ref, wb0, bb0_, wb1, bb1_, wb2, bb2_, wcb, bcb_,
                 wlb, blb_, wrc, brc, wrl, brl,
                 ns, nl, nr, npp, out_ref):
    norms = (ns, nl, nr, npp)
    for e in range(BB):
        xe = x_ref[e]
        h = _leaky(_gcn(xe, wb0, bb0_, norms))
        h = _leaky(_gcn(h, wb1, bb1_, norms))
        h = _leaky(_gcn(h, wb2, bb2_, norms))
        hc = _leaky(_gcn(h, wcb, bcb_, norms))
        hl = _leaky(_gcn(h, wlb, blb_, norms))
        flat_c = _head_flat(hc, wrc, brc, norms)
        flat_l = _head_flat(hl, wrl, brl, norms)

        stage = x_ref[e, 1023:1024, 0:1].astype(jnp.int32)
        m0 = (stage == 0).astype(jnp.float32)
        m1 = (stage == 1).astype(jnp.float32)
        m2 = (stage == 2).astype(jnp.float32)
        m3 = (stage == 3).astype(jnp.float32)

        out_ref[0, LEAF_IDX:FEAT_IDX, e:e + 1] = m1 * flat_l
        out_ref[0, FEAT_IDX:THR_IDX, e:e + 1] = jnp.broadcast_to(
            m2, (THR_IDX - FEAT_IDX, 1))
        out_ref[0, THR_IDX:OP_IDX, e:e + 1] = jnp.broadcast_to(
            m3, (OP_IDX - THR_IDX, 1))
        out_ref[0, OP_IDX:EOS_IDX, e:e + 1] = m0 * flat_c
        out_ref[0, EOS_IDX:P_DIM, e:e + 1] = m0


def kernel(x, Wb0, bb0, Wb1, bb1, Wb2, bb2, Wcb, bcb, Wch, bch,
           Wlb, blb, Wlh, blh):
    bb0r = bb0.reshape(1, HID)
    bb1r = bb1.reshape(1, HID)
    bb2r = bb2.reshape(1, HID)
    bcbr = bcb.reshape(1, HID)
    blbr = blb.reshape(1, HID)
    wrc = jnp.tile(Wch.T, (MAX_NODES, 1))
    brc = jnp.tile(bch.reshape(2, 1), (MAX_NODES, 1))
    wrl = jnp.tile(Wlh.T, (MAX_NODES, 1))
    brl = jnp.tile(blh.reshape(2, 1), (MAX_NODES, 1))

    full = lambda s: pl.BlockSpec(s, lambda i: tuple(0 for _ in s))
    grid = (B // BB,)
    out_t = pl.pallas_call(
        _kernel_body,
        grid=grid,
        in_specs=[
            pl.BlockSpec((BB, MAX_NODES + 1, N_FEAT), lambda i: (i, 0, 0)),
            full((N_FEAT, HID)), full((1, HID)),
            full((HID, HID)), full((1, HID)),
            full((HID, HID)), full((1, HID)),
            full((HID, HID)), full((1, HID)),
            full((HID, HID)), full((1, HID)),
            full((2 * MAX_NODES, HID)), full((2 * MAX_NODES, 1)),
            full((2 * MAX_NODES, HID)), full((2 * MAX_NODES, 1)),
            full((MAX_NODES + 1, 1)), full((511, 1)),
            full((511, 1)), full((1022, 1)),
        ],
        out_specs=pl.BlockSpec((1, P_DIM, BB), lambda i: (i, 0, 0)),
        out_shape=jax.ShapeDtypeStruct((B // BB, P_DIM, BB), jnp.float32),
        compiler_params=pltpu.CompilerParams(
            dimension_semantics=("arbitrary",)),
    )(x, Wb0, bb0r, Wb1, bb1r, Wb2, bb2r, Wcb, bcbr, Wlb, blbr,
      wrc, brc, wrl, brl,
      jnp.asarray(NS_COL), jnp.asarray(NL_COL),
      jnp.asarray(NR_COL), jnp.asarray(NPAR_COL))
    return out_t.transpose(0, 2, 1).reshape(B, P_DIM)
